# Initial kernel scaffold; baseline (speedup 1.0000x reference)
import jax
import jax.numpy as jnp
from jax.experimental import pallas as pl


def kernel(features, edge_index, W1, W2):
    raise NotImplementedError("write your pallas kernel here")


# speedup vs baseline: 8.5808x; 8.5808x over previous
import functools

import jax
import jax.numpy as jnp
from jax import lax
from jax.experimental import pallas as pl
from jax.experimental.pallas import tpu as pltpu
from jax.experimental.pallas import tpu_sc as plsc

N_NODES = 10000
N_EDGES = 320000
IN_DIM = 128
HIDDEN_DIM = 128
OUT_DIM = 64
NLAYERS = 2

NC = 2
NS = 16
NW = NC * NS
EDGES_PER_W = N_EDGES // NW
CHUNK = 125
CHUNKS_PER_W = EDGES_PER_W // CHUNK
STRIPE = 624
LAST_STRIPE = N_NODES - 15 * STRIPE


def _mlp_body(x_ref, w1_ref, w2_ref, o_ref):
    h = jnp.dot(x_ref[...], w1_ref[...], preferred_element_type=jnp.float32)
    h = jnp.maximum(h, 0.0)
    o_ref[...] = jnp.dot(h, w2_ref[...], preferred_element_type=jnp.float32)


_mlp = pl.pallas_call(
    _mlp_body,
    grid=(10,),
    in_specs=[
        pl.BlockSpec((N_NODES // 10, IN_DIM), lambda i: (i, 0)),
        pl.BlockSpec((IN_DIM, HIDDEN_DIM), lambda i: (0, 0)),
        pl.BlockSpec((HIDDEN_DIM, OUT_DIM), lambda i: (0, 0)),
    ],
    out_specs=pl.BlockSpec((N_NODES // 10, OUT_DIM), lambda i: (i, 0)),
    out_shape=jax.ShapeDtypeStruct((N_NODES, OUT_DIM), jnp.float32),
)


def _add_body(a_ref, b_ref, o_ref):
    o_ref[...] = a_ref[...] + b_ref[...]


_add_halves = pl.pallas_call(
    _add_body,
    grid=(10,),
    in_specs=[
        pl.BlockSpec((N_NODES // 10, OUT_DIM), lambda i: (i, 0)),
        pl.BlockSpec((N_NODES // 10, OUT_DIM), lambda i: (i + 10, 0)),
    ],
    out_specs=pl.BlockSpec((N_NODES // 10, OUT_DIM), lambda i: (i, 0)),
    out_shape=jax.ShapeDtypeStruct((N_NODES, OUT_DIM), jnp.float32),
)


_sc_mesh = plsc.VectorSubcoreMesh(
    core_axis_name="c", subcore_axis_name="s", num_cores=NC, num_subcores=NS
)


@functools.partial(
    pl.kernel,
    out_type=jax.ShapeDtypeStruct((NC * N_NODES, OUT_DIM), jnp.float32),
    mesh=_sc_mesh,
    compiler_params=pltpu.CompilerParams(use_tc_tiling_on_sc=False),
    scratch_types=[
        pltpu.VMEM((CHUNKS_PER_W, CHUNK), jnp.int32),
        pltpu.VMEM((CHUNKS_PER_W, CHUNK), jnp.int32),
        pltpu.VMEM((CHUNK, OUT_DIM), jnp.float32),
        pltpu.VMEM((LAST_STRIPE, OUT_DIM), jnp.float32),
        pltpu.VMEM_SHARED((N_NODES, OUT_DIM), jnp.float32),
    ],
)
def _propagate(h_hbm, src_hbm, dst_hbm, out_hbm, src_v, dst_v, rows_v, zbuf_v, acc_sh):
    cid = lax.axis_index("c")
    sid = lax.axis_index("s")
    wid = cid * NS + sid

    zeros16 = jnp.zeros((16,), jnp.float32)

    def _zero_row(r, _):
        for l in range(OUT_DIM // 16):
            zbuf_v[r, pl.ds(l * 16, 16)] = zeros16
        return 0

    lax.fori_loop(0, LAST_STRIPE, _zero_row, 0)

    @pl.when(sid < NS - 1)
    def _():
        pltpu.sync_copy(zbuf_v.at[pl.ds(0, STRIPE)],
                        acc_sh.at[pl.ds(sid * STRIPE, STRIPE)])

    @pl.when(sid == NS - 1)
    def _():
        pltpu.sync_copy(zbuf_v, acc_sh.at[pl.ds((NS - 1) * STRIPE, LAST_STRIPE)])

    plsc.subcore_barrier()

    pltpu.sync_copy(src_hbm.at[pl.ds(wid * CHUNKS_PER_W, CHUNKS_PER_W)], src_v)
    pltpu.sync_copy(dst_hbm.at[pl.ds(wid * CHUNKS_PER_W, CHUNKS_PER_W)], dst_v)

    def _edge_chunk(ci, _):
        pltpu.sync_copy(h_hbm.at[src_v.at[ci]], rows_v)
        pltpu.sync_copy(rows_v, acc_sh.at[dst_v.at[ci]], add=True)
        return 0

    lax.fori_loop(0, CHUNKS_PER_W, _edge_chunk, 0)
    plsc.subcore_barrier()

    @pl.when(sid < NS - 1)
    def _():
        pltpu.sync_copy(
            acc_sh.at[pl.ds(sid * STRIPE, STRIPE)],
            out_hbm.at[pl.ds(cid * N_NODES + sid * STRIPE, STRIPE)],
        )

    @pl.when(sid == NS - 1)
    def _():
        pltpu.sync_copy(
            acc_sh.at[pl.ds((NS - 1) * STRIPE, LAST_STRIPE)],
            out_hbm.at[pl.ds(cid * N_NODES + (NS - 1) * STRIPE, LAST_STRIPE)],
        )


@jax.jit
def kernel(features, edge_index, W1, W2):
    h = _mlp(features, W1, W2)
    edges = edge_index.astype(jnp.int32).reshape(2, N_EDGES // CHUNK, CHUNK)
    src = edges[0]
    dst = edges[1]
    for _ in range(NLAYERS):
        partials = _propagate(h, src, dst)
        h = _add_halves(partials, partials)
    return h


# speedup vs baseline: 12.0159x; 1.4003x over previous
import functools

import jax
import jax.numpy as jnp
from jax import lax
from jax.experimental import pallas as pl
from jax.experimental.pallas import tpu as pltpu
from jax.experimental.pallas import tpu_sc as plsc

N_NODES = 10000
N_EDGES = 320000
IN_DIM = 128
HIDDEN_DIM = 128
OUT_DIM = 64
NLAYERS = 2

NC = 2
NS = 16
NW = NC * NS
EDGES_PER_W = N_EDGES // NW
CHUNK = 125
CHUNKS_PER_W = EDGES_PER_W // CHUNK
STRIPE = 624
LAST_STRIPE = N_NODES - 15 * STRIPE


def _mlp_body(x_ref, w1_ref, w2_ref, o_ref):
    h = jnp.dot(x_ref[...], w1_ref[...], preferred_element_type=jnp.float32)
    h = jnp.maximum(h, 0.0)
    o_ref[...] = jnp.dot(h, w2_ref[...], preferred_element_type=jnp.float32)


_mlp = pl.pallas_call(
    _mlp_body,
    grid=(10,),
    in_specs=[
        pl.BlockSpec((N_NODES // 10, IN_DIM), lambda i: (i, 0)),
        pl.BlockSpec((IN_DIM, HIDDEN_DIM), lambda i: (0, 0)),
        pl.BlockSpec((HIDDEN_DIM, OUT_DIM), lambda i: (0, 0)),
    ],
    out_specs=pl.BlockSpec((N_NODES // 10, OUT_DIM), lambda i: (i, 0)),
    out_shape=jax.ShapeDtypeStruct((N_NODES, OUT_DIM), jnp.float32),
)


def _add_body(a_ref, b_ref, o_ref):
    o_ref[...] = a_ref[...] + b_ref[...]


_add_halves = pl.pallas_call(
    _add_body,
    grid=(10,),
    in_specs=[
        pl.BlockSpec((N_NODES // 10, OUT_DIM), lambda i: (i, 0)),
        pl.BlockSpec((N_NODES // 10, OUT_DIM), lambda i: (i + 10, 0)),
    ],
    out_specs=pl.BlockSpec((N_NODES // 10, OUT_DIM), lambda i: (i, 0)),
    out_shape=jax.ShapeDtypeStruct((N_NODES, OUT_DIM), jnp.float32),
)


_sc_mesh = plsc.VectorSubcoreMesh(
    core_axis_name="c", subcore_axis_name="s", num_cores=NC, num_subcores=NS
)


@functools.partial(
    pl.kernel,
    out_type=jax.ShapeDtypeStruct((NC * N_NODES, OUT_DIM), jnp.float32),
    mesh=_sc_mesh,
    compiler_params=pltpu.CompilerParams(use_tc_tiling_on_sc=False),
    scratch_types=[
        pltpu.VMEM((CHUNKS_PER_W, CHUNK), jnp.int32),
        pltpu.VMEM((CHUNKS_PER_W, CHUNK), jnp.int32),
        pltpu.VMEM((2, CHUNK, OUT_DIM), jnp.float32),
        pltpu.VMEM((LAST_STRIPE, OUT_DIM), jnp.float32),
        pltpu.VMEM_SHARED((N_NODES, OUT_DIM), jnp.float32),
        pltpu.SemaphoreType.DMA,
        pltpu.SemaphoreType.DMA,
    ],
)
def _propagate(h_hbm, src_hbm, dst_hbm, out_hbm, src_v, dst_v, rows_v, zbuf_v,
               acc_sh, semg0, semg1):
    cid = lax.axis_index("c")
    sid = lax.axis_index("s")
    wid = cid * NS + sid

    zeros16 = jnp.zeros((16,), jnp.float32)

    def _zero_row(r, _):
        for l in range(OUT_DIM // 16):
            zbuf_v[r, pl.ds(l * 16, 16)] = zeros16
        return 0

    lax.fori_loop(0, LAST_STRIPE, _zero_row, 0)

    @pl.when(sid < NS - 1)
    def _():
        pltpu.sync_copy(zbuf_v.at[pl.ds(0, STRIPE)],
                        acc_sh.at[pl.ds(sid * STRIPE, STRIPE)])

    @pl.when(sid == NS - 1)
    def _():
        pltpu.sync_copy(zbuf_v, acc_sh.at[pl.ds((NS - 1) * STRIPE, LAST_STRIPE)])

    plsc.subcore_barrier()

    pltpu.sync_copy(src_hbm.at[pl.ds(wid * CHUNKS_PER_W, CHUNKS_PER_W)], src_v)
    pltpu.sync_copy(dst_hbm.at[pl.ds(wid * CHUNKS_PER_W, CHUNKS_PER_W)], dst_v)

    sems = (semg0, semg1)

    def _gather(ci, b):
        return pltpu.make_async_copy(h_hbm.at[src_v.at[ci]], rows_v.at[b], sems[b])

    for b in range(2):
        _gather(b, b).start()

    def _pair(k, _):
        i0 = k * 2
        for b in range(2):
            ci = i0 + b
            _gather(ci, b).wait()
            pltpu.sync_copy(rows_v.at[b], acc_sh.at[dst_v.at[ci]], add=True)

            @pl.when(ci + 2 < CHUNKS_PER_W)
            def _():
                _gather(ci + 2, b).start()

        return 0

    lax.fori_loop(0, CHUNKS_PER_W // 2, _pair, 0)
    plsc.subcore_barrier()

    @pl.when(sid < NS - 1)
    def _():
        pltpu.sync_copy(
            acc_sh.at[pl.ds(sid * STRIPE, STRIPE)],
            out_hbm.at[pl.ds(cid * N_NODES + sid * STRIPE, STRIPE)],
        )

    @pl.when(sid == NS - 1)
    def _():
        pltpu.sync_copy(
            acc_sh.at[pl.ds((NS - 1) * STRIPE, LAST_STRIPE)],
            out_hbm.at[pl.ds(cid * N_NODES + (NS - 1) * STRIPE, LAST_STRIPE)],
        )


@jax.jit
def kernel(features, edge_index, W1, W2):
    h = _mlp(features, W1, W2)
    edges = edge_index.astype(jnp.int32).reshape(2, N_EDGES // CHUNK, CHUNK)
    src = edges[0]
    dst = edges[1]
    for _ in range(NLAYERS):
        partials = _propagate(h, src, dst)
        h = _add_halves(partials, partials)
    return h


# speedup vs baseline: 13.6876x; 1.1391x over previous
import functools

import jax
import jax.numpy as jnp
from jax import lax
from jax.experimental import pallas as pl
from jax.experimental.pallas import tpu as pltpu
from jax.experimental.pallas import tpu_sc as plsc

N_NODES = 10000
N_EDGES = 320000
IN_DIM = 128
HIDDEN_DIM = 128
OUT_DIM = 64
NLAYERS = 2

NC = 2
NS = 16
NW = NC * NS
EDGES_PER_W = N_EDGES // NW
CHUNK = 125
CHUNKS_PER_W = EDGES_PER_W // CHUNK
ROWS_PER_TILE = N_NODES // NS
NBUF = 4


def _mlp_body(x_ref, w1_ref, w2_ref, o_ref):
    h = jnp.dot(x_ref[...], w1_ref[...], preferred_element_type=jnp.float32)
    h = jnp.maximum(h, 0.0)
    o_ref[...] = jnp.dot(h, w2_ref[...], preferred_element_type=jnp.float32)


_mlp = pl.pallas_call(
    _mlp_body,
    grid=(10,),
    in_specs=[
        pl.BlockSpec((N_NODES // 10, IN_DIM), lambda i: (i, 0)),
        pl.BlockSpec((IN_DIM, HIDDEN_DIM), lambda i: (0, 0)),
        pl.BlockSpec((HIDDEN_DIM, OUT_DIM), lambda i: (0, 0)),
    ],
    out_specs=pl.BlockSpec((N_NODES // 10, OUT_DIM), lambda i: (i, 0)),
    out_shape=jax.ShapeDtypeStruct((N_NODES, OUT_DIM), jnp.float32),
)


def _add_body(a_ref, b_ref, o_ref):
    o_ref[...] = a_ref[...] + b_ref[...]


_add_halves = pl.pallas_call(
    _add_body,
    grid=(10,),
    in_specs=[
        pl.BlockSpec((N_NODES // 10, OUT_DIM), lambda i: (i, 0)),
        pl.BlockSpec((N_NODES // 10, OUT_DIM), lambda i: (i + 10, 0)),
    ],
    out_specs=pl.BlockSpec((N_NODES // 10, OUT_DIM), lambda i: (i, 0)),
    out_shape=jax.ShapeDtypeStruct((N_NODES, OUT_DIM), jnp.float32),
)


_sc_mesh = plsc.VectorSubcoreMesh(
    core_axis_name="c", subcore_axis_name="s", num_cores=NC, num_subcores=NS
)


@functools.partial(
    pl.kernel,
    out_type=jax.ShapeDtypeStruct((NC * N_NODES, OUT_DIM), jnp.float32),
    mesh=_sc_mesh,
    compiler_params=pltpu.CompilerParams(use_tc_tiling_on_sc=False),
    scratch_types=[
        pltpu.VMEM((CHUNKS_PER_W, CHUNK), jnp.int32),
        pltpu.VMEM((CHUNKS_PER_W, CHUNK), jnp.int32),
        pltpu.VMEM((NBUF, CHUNK, OUT_DIM), jnp.float32),
        pltpu.VMEM_SHARED((N_NODES, OUT_DIM), jnp.float32),
        pltpu.SemaphoreType.DMA,
        pltpu.SemaphoreType.DMA,
        pltpu.SemaphoreType.DMA,
        pltpu.SemaphoreType.DMA,
    ],
)
def _propagate(h_hbm, src_hbm, dst_hbm, out_hbm, src_v, dst_v, rows_v,
               acc_sh, gsem0, gsem1, gsem2, gsem3):
    cid = lax.axis_index("c")
    sid = lax.axis_index("s")
    wid = cid * NS + sid

    zeros16 = jnp.zeros((16,), jnp.float32)

    def _zero_row(r, _):
        for b in range(NBUF):
            for l in range(OUT_DIM // 16):
                rows_v[b, r, pl.ds(l * 16, 16)] = zeros16
        return 0

    lax.fori_loop(0, CHUNK, _zero_row, 0)
    for k in range(ROWS_PER_TILE // CHUNK):
        pltpu.sync_copy(
            rows_v.at[k % NBUF],
            acc_sh.at[pl.ds(sid * ROWS_PER_TILE + k * CHUNK, CHUNK)],
        )
    plsc.subcore_barrier()

    pltpu.sync_copy(src_hbm.at[pl.ds(wid * CHUNKS_PER_W, CHUNKS_PER_W)], src_v)
    pltpu.sync_copy(dst_hbm.at[pl.ds(wid * CHUNKS_PER_W, CHUNKS_PER_W)], dst_v)

    gsems = (gsem0, gsem1, gsem2, gsem3)

    def _gather(ci, b):
        return pltpu.make_async_copy(h_hbm.at[src_v.at[ci]], rows_v.at[b],
                                     gsems[b])

    for b in range(3):
        _gather(b, b).start()

    def _quad(k, _):
        i0 = k * 4
        for b in range(4):
            ci = i0 + b
            _gather(ci, b).wait()
            pltpu.sync_copy(rows_v.at[b], acc_sh.at[dst_v.at[ci]], add=True)

            @pl.when(ci + 3 < CHUNKS_PER_W)
            def _():
                _gather(ci + 3, (b + 3) % 4).start()

        return 0

    lax.fori_loop(0, CHUNKS_PER_W // 4, _quad, 0)
    plsc.subcore_barrier()

    pltpu.sync_copy(
        acc_sh.at[pl.ds(sid * ROWS_PER_TILE, ROWS_PER_TILE)],
        out_hbm.at[pl.ds(cid * N_NODES + sid * ROWS_PER_TILE, ROWS_PER_TILE)],
    )


@jax.jit
def kernel(features, edge_index, W1, W2):
    h = _mlp(features, W1, W2)
    edges = edge_index.astype(jnp.int32).reshape(2, N_EDGES // CHUNK, CHUNK)
    src = edges[0]
    dst = edges[1]
    for _ in range(NLAYERS):
        partials = _propagate(h, src, dst)
        h = _add_halves(partials, partials)
    return h
